# Initial kernel scaffold; baseline (speedup 1.0000x reference)
import jax
import jax.numpy as jnp
from jax.experimental import pallas as pl


def kernel(input_ids, prompt_ids, embed_weight):
    raise NotImplementedError("write your pallas kernel here")


# speedup vs baseline: 1.7423x; 1.7423x over previous
import jax
import jax.numpy as jnp
from jax import lax
from jax.experimental import pallas as pl
from jax.experimental.pallas import tpu as pltpu
from jax.experimental.pallas import tpu_sc as plsc

_NC = 2
_NS = 16
_NW = _NC * _NS


def _make_sc_kernel(B, L, P, D):
    rows_per_w = P // _NW
    mesh = plsc.VectorSubcoreMesh(core_axis_name="c", subcore_axis_name="s")

    def body(input_ids_hbm, prompt_hbm, idx_hbm, table_hbm,
             ids_out_hbm, emb_out_hbm,
             idx_v, rows_v, ids_v, gsem, wsem):
        wid = lax.axis_index("s") * _NC + lax.axis_index("c")
        base = wid * rows_per_w
        pltpu.sync_copy(idx_hbm.at[pl.ds(base, rows_per_w)], idx_v)
        gather = pltpu.async_copy(table_hbm.at[idx_v], rows_v, gsem)

        @pl.when(wid < B)
        def _():
            pltpu.sync_copy(prompt_hbm, ids_v)
            pltpu.sync_copy(ids_v, ids_out_hbm.at[wid, pl.ds(0, P)])
            pltpu.sync_copy(input_ids_hbm.at[wid], ids_v)
            pltpu.sync_copy(ids_v, ids_out_hbm.at[wid, pl.ds(P, L)])

        gather.wait()
        writes = [
            pltpu.async_copy(
                rows_v, emb_out_hbm.at[b, pl.ds(base, rows_per_w)], wsem)
            for b in range(B)
        ]
        for w in writes:
            w.wait()

    return pl.kernel(
        body,
        out_type=(
            jax.ShapeDtypeStruct((B, P + L), jnp.int32),
            jax.ShapeDtypeStruct((B, P, D), jnp.float32),
        ),
        mesh=mesh,
        scratch_types=[
            pltpu.VMEM((rows_per_w,), jnp.int32),
            pltpu.VMEM((rows_per_w, D), jnp.float32),
            pltpu.VMEM((max(P, L),), jnp.int32),
            pltpu.SemaphoreType.DMA,
            pltpu.SemaphoreType.DMA,
        ],
    )


def kernel(input_ids, prompt_ids, embed_weight):
    B, L = input_ids.shape
    P, D = embed_weight.shape
    idx = jnp.where(prompt_ids < P, prompt_ids, P).astype(jnp.int32)
    table_aug = jnp.concatenate(
        [embed_weight, jnp.full((1, D), jnp.nan, embed_weight.dtype)], axis=0)
    sc = _make_sc_kernel(B, L, P, D)
    prepended_ids, prompt_embeds = sc(input_ids, prompt_ids, idx, table_aug)
    return prepended_ids, prompt_embeds
